# baseline (speedup 1.0000x reference)
import functools

import jax
import jax.numpy as jnp
from jax import lax
from jax.experimental import pallas as pl
from jax.experimental.pallas import tpu as pltpu
from jax.experimental.pallas import tpu_sc as plsc


def _sum_lanes_i16(a):
    n = a.shape[1]
    while n > 128:
        n //= 2
        a = a[:, :n] + a[:, n:]
    return jnp.sum(a.astype(jnp.int32), axis=-1, keepdims=True)


def _greedy_multi(datas, kth, limits, nbits, bias):
    kth32 = jnp.int32(kth)
    ps = [jnp.zeros((d.shape[0], 1), jnp.int32) for d in datas]
    cs = [jnp.zeros((d.shape[0], 1), jnp.int32) for d in datas]
    for i in range(nbits):
        bit = 1 << (nbits - 1 - i)
        for g, data in enumerate(datas):
            cand = jnp.bitwise_or(ps[g], bit)
            trial = (cand - bias).astype(jnp.int16)
            c = _sum_lanes_i16((data < trial).astype(jnp.int16))
            accept = c + limits[g] <= kth32
            ps[g] = jnp.where(accept, cand, ps[g])
            cs[g] = jnp.where(accept, c, cs[g])
    return ps, cs


def _median_body(x_ref, val_ref, idx_ref, *, kth, groups):
    i32min = jnp.int32(-(2 ** 31))
    i16max = jnp.int16(2 ** 15 - 1)
    xb = x_ref[...]
    rows, n = xb.shape
    gr = rows // groups
    s = lax.bitcast_convert_type(xb, jnp.int32)
    key = jnp.where(s >= 0, s, jnp.bitwise_xor(jnp.bitwise_not(s), i32min))
    hi = jnp.right_shift(key, 16).astype(jnp.int16)
    lo = (jnp.bitwise_and(key, 0xFFFF) - 32768).astype(jnp.int16)

    his = [hi[g * gr:(g + 1) * gr] for g in range(groups)]
    los = [lo[g * gr:(g + 1) * gr] for g in range(groups)]
    zeros = [jnp.zeros((gr, 1), jnp.int32) for _ in range(groups)]

    hps, c1s = _greedy_multi(his, kth, zeros, 16, 32768)

    m1s = [h == (hp - 32768).astype(jnp.int16) for h, hp in zip(his, hps)]
    loxs = [jnp.where(m, l, i16max) for m, l in zip(m1s, los)]
    lps, c2s = _greedy_multi(loxs, kth, c1s, 16, 32768)

    iota = lax.broadcasted_iota(jnp.int16, (gr, n), 1)
    ioxs = [jnp.where(jnp.logical_and(m, l == (lp - 32768).astype(jnp.int16)),
                      iota, i16max)
            for m, l, lp in zip(m1s, los, lps)]
    limits3 = [c1 + c2 for c1, c2 in zip(c1s, c2s)]
    idxs, _ = _greedy_multi(ioxs, kth, limits3, 11, 0)

    for g in range(groups):
        v = jnp.bitwise_or(jnp.left_shift(hps[g] - 32768, 16), lps[g])
        sv = jnp.where(v >= 0, v,
                       jnp.bitwise_not(jnp.bitwise_xor(v, i32min)))
        val_ref[g * gr:(g + 1) * gr, :] = lax.bitcast_convert_type(
            sv, jnp.float32)
        idx_ref[g * gr:(g + 1) * gr, :] = idxs[g]


def _median_2d(x2, block_rows):
    m, n = x2.shape
    kth = (n - 1) // 2
    grid = (m // block_rows,)
    vals, idx = pl.pallas_call(
        functools.partial(_median_body, kth=kth, groups=2),
        grid=grid,
        in_specs=[pl.BlockSpec((block_rows, n), lambda j: (j, 0))],
        out_specs=[
            pl.BlockSpec((block_rows, 1), lambda j: (j, 0)),
            pl.BlockSpec((block_rows, 1), lambda j: (j, 0)),
        ],
        out_shape=[
            jax.ShapeDtypeStruct((m, 1), jnp.float32),
            jax.ShapeDtypeStruct((m, 1), jnp.int32),
        ],
    )(x2)
    return vals[:, 0], idx[:, 0]


_N = 2048
_KTH = (_N - 1) // 2
_NVEC = _N // 16
_CHUNK = 16


def _keys16(xv):
    i32min = jnp.int32(-(2 ** 31))
    s = plsc.bitcast(xv, jnp.int32)
    return jnp.where(s >= 0, s, jnp.bitwise_xor(jnp.bitwise_not(s), i32min))


def _make_sc_median(m_sc):
    info = plsc.get_sparse_core_info()
    nworkers = info.num_cores * info.num_subcores
    rpw = m_sc // nworkers
    assert m_sc % (nworkers * _CHUNK) == 0
    mesh = plsc.VectorSubcoreMesh(core_axis_name="c", subcore_axis_name="s")

    @functools.partial(
        pl.kernel,
        mesh=mesh,
        compiler_params=pltpu.CompilerParams(needs_layout_passes=False),
        out_type=[
            jax.ShapeDtypeStruct((m_sc,), jnp.float32),
            jax.ShapeDtypeStruct((m_sc,), jnp.int32),
        ],
        scratch_types=[
            pltpu.VMEM((_CHUNK * _N,), jnp.float32),
            pltpu.VMEM((512,), jnp.int32),
            pltpu.VMEM((_N + 16,), jnp.int32),
            pltpu.VMEM((_N + 16,), jnp.int32),
            pltpu.VMEM((rpw,), jnp.float32),
            pltpu.VMEM((rpw,), jnp.int32),
        ],
    )
    def sc_median(x_hbm, val_hbm, idx_hbm, xbuf, hist, ckey, cidx, oval, oidx):
        wid = lax.axis_index("s") * info.num_cores + lax.axis_index("c")
        base = wid * rpw
        zeros16 = jnp.zeros((16,), jnp.int32)
        ones16 = jnp.ones((16,), jnp.int32)
        lane = lax.iota(jnp.int32, 16)

        def zero_hist(j, _):
            hist[pl.ds(j * 16, 16)] = zeros16
            return 0

        lax.fori_loop(0, 512 // 16, zero_hist, 0)

        def do_chunk(ci, _):
            pltpu.sync_copy(
                x_hbm.at[pl.ds((base + ci * _CHUNK) * _N, _CHUNK * _N)],
                xbuf)

            def do_row(ri, acc):
                def pass_a(j4, _):
                    for u in range(4):
                        key = _keys16(
                            xbuf[pl.ds(ri * _N + (j4 * 4 + u) * 16, 16)])
                        b = jnp.right_shift(key, 23) + 256
                        plsc.addupdate_scatter(hist, [b], ones16)
                    return 0

                lax.fori_loop(0, _NVEC // 4, pass_a, 0)

                def locate(j, carry):
                    tot, bfound, cbelow = carry
                    hv = hist[pl.ds(j * 16, 16)]
                    hist[pl.ds(j * 16, 16)] = zeros16
                    cs = plsc.cumsum(hv)
                    cross = (tot + cs) > _KTH
                    has = jnp.any(cross)
                    first = jnp.max(plsc.all_reduce_ffs(cross))
                    excl = tot + cs - hv
                    cb = jnp.sum(jnp.where(lane == first, excl, 0))
                    newly = jnp.logical_and(has, bfound < 0)
                    bfound = jnp.where(newly, j * 16 + first, bfound)
                    cbelow = jnp.where(newly, cb, cbelow)
                    return tot + jnp.max(cs), bfound, cbelow

                _, bkt, cbelow = lax.fori_loop(
                    0, 512 // 16, locate,
                    (jnp.int32(0), jnp.int32(-1), jnp.int32(0)))
                b9 = bkt - 256

                def pass_b(j, cnt):
                    key = _keys16(xbuf[pl.ds(ri * _N + j * 16, 16)])
                    mask = jnp.right_shift(key, 23) == b9
                    plsc.store_compressed(ckey.at[pl.ds(cnt, 16)], key, mask=mask)
                    iv = j * 16 + lane
                    plsc.store_compressed(cidx.at[pl.ds(cnt, 16)], iv, mask=mask)
                    return cnt + jnp.max(
                        plsc.all_reduce_population_count(mask))

                cnt = lax.fori_loop(0, _NVEC, pass_b, jnp.int32(0))
                ckey[pl.ds(cnt, 16)] = jnp.full((16,), 2 ** 31 - 1, jnp.int32)
                nv = (cnt + 15) // 16

                bktbase = jnp.left_shift(b9, 23)

                def count_lt(trial):
                    def cbody(v, c):
                        kv = ckey[pl.ds(v * 16, 16)]
                        return c + jnp.sum((kv < trial).astype(jnp.int32))
                    return lax.fori_loop(0, nv, cbody, jnp.int32(0))

                p = jnp.int32(0)
                c_acc = jnp.int32(0)
                for bitpos in range(22, -1, -1):
                    cand = jnp.bitwise_or(p, jnp.int32(1 << bitpos))
                    c = count_lt(bktbase + cand)
                    ok = cbelow + c <= _KTH
                    p = jnp.where(ok, cand, p)
                    c_acc = jnp.where(ok, c, c_acc)
                vkey = bktbase + p
                r_total = _KTH - cbelow - c_acc

                def idx_scan(v, carry):
                    cum, fidx = carry
                    kv = ckey[pl.ds(v * 16, 16)]
                    iv = cidx[pl.ds(v * 16, 16)]
                    eq = kv == vkey
                    cs = plsc.cumsum(eq.astype(jnp.int32))
                    hitm = jnp.logical_and(eq, cs == r_total - cum + 1)
                    fidx = fidx + jnp.sum(jnp.where(hitm, iv, 0))
                    return cum + jnp.sum(eq.astype(jnp.int32)), fidx

                _, med_idx = lax.fori_loop(0, nv, idx_scan,
                                           (jnp.int32(0), jnp.int32(0)))
                mv16 = xbuf[pl.ds(ri * _N + med_idx - med_idx % 16, 16)]
                med_val = jnp.sum(jnp.where(lane == med_idx % 16, mv16, 0.0))
                vvec, ivec = acc
                sel = lane == ri
                vvec = jnp.where(sel, jnp.full((16,), med_val, jnp.float32),
                                 vvec)
                ivec = jnp.where(sel, jnp.full((16,), med_idx, jnp.int32),
                                 ivec)
                return vvec, ivec

            vvec, ivec = lax.fori_loop(
                0, _CHUNK, do_row,
                (jnp.zeros((16,), jnp.float32), jnp.zeros((16,), jnp.int32)))
            oval[pl.ds(ci * _CHUNK, 16)] = vvec
            oidx[pl.ds(ci * _CHUNK, 16)] = ivec
            return 0

        lax.fori_loop(0, rpw // _CHUNK, do_chunk, 0)
        pltpu.sync_copy(oval, val_hbm.at[pl.ds(base, rpw)])
        pltpu.sync_copy(oidx, idx_hbm.at[pl.ds(base, rpw)])

    return sc_median


_M_SC = 2560


def kernel(x):
    b, s, n = x.shape
    m = b * s
    x2 = x.reshape(m, n)
    if m > _M_SC and (m - _M_SC) % 256 == 0:
        m_sc = _M_SC
        val_sc, idx_sc = _make_sc_median(m_sc)(x2[:m_sc].reshape(-1))
        val_tc, idx_tc = _median_2d(x2[m_sc:], 256)
        vals = jnp.concatenate([val_sc, val_tc])
        idx = jnp.concatenate([idx_sc, idx_tc])
    else:
        block_rows = 256 if m % 256 == 0 else m
        vals, idx = _median_2d(x2, block_rows)
    return vals.reshape(b, s), idx.reshape(b, s).astype(jnp.int64)


# speedup vs baseline: 1.0242x; 1.0242x over previous
import functools

import jax
import jax.numpy as jnp
from jax import lax
from jax.experimental import pallas as pl
from jax.experimental.pallas import tpu as pltpu
from jax.experimental.pallas import tpu_sc as plsc


def _sum_lanes_i16(a):
    n = a.shape[1]
    while n > 128:
        n //= 2
        a = a[:, :n] + a[:, n:]
    return jnp.sum(a.astype(jnp.int32), axis=-1, keepdims=True)


def _greedy_multi(datas, kth, limits, nbits, bias):
    kth32 = jnp.int32(kth)
    ps = [jnp.zeros((d.shape[0], 1), jnp.int32) for d in datas]
    cs = [jnp.zeros((d.shape[0], 1), jnp.int32) for d in datas]
    for i in range(nbits):
        bit = 1 << (nbits - 1 - i)
        for g, data in enumerate(datas):
            cand = jnp.bitwise_or(ps[g], bit)
            trial = (cand - bias).astype(jnp.int16)
            c = _sum_lanes_i16((data < trial).astype(jnp.int16))
            accept = c + limits[g] <= kth32
            ps[g] = jnp.where(accept, cand, ps[g])
            cs[g] = jnp.where(accept, c, cs[g])
    return ps, cs


def _median_body(x_ref, val_ref, idx_ref, *, kth, groups):
    i32min = jnp.int32(-(2 ** 31))
    i16max = jnp.int16(2 ** 15 - 1)
    xb = x_ref[...]
    rows, n = xb.shape
    gr = rows // groups
    s = lax.bitcast_convert_type(xb, jnp.int32)
    key = jnp.where(s >= 0, s, jnp.bitwise_xor(jnp.bitwise_not(s), i32min))
    hi = jnp.right_shift(key, 16).astype(jnp.int16)
    lo = (jnp.bitwise_and(key, 0xFFFF) - 32768).astype(jnp.int16)

    his = [hi[g * gr:(g + 1) * gr] for g in range(groups)]
    los = [lo[g * gr:(g + 1) * gr] for g in range(groups)]
    zeros = [jnp.zeros((gr, 1), jnp.int32) for _ in range(groups)]

    hps, c1s = _greedy_multi(his, kth, zeros, 16, 32768)

    m1s = [h == (hp - 32768).astype(jnp.int16) for h, hp in zip(his, hps)]
    loxs = [jnp.where(m, l, i16max) for m, l in zip(m1s, los)]
    lps, c2s = _greedy_multi(loxs, kth, c1s, 16, 32768)

    iota = lax.broadcasted_iota(jnp.int16, (gr, n), 1)
    ioxs = [jnp.where(jnp.logical_and(m, l == (lp - 32768).astype(jnp.int16)),
                      iota, i16max)
            for m, l, lp in zip(m1s, los, lps)]
    limits3 = [c1 + c2 for c1, c2 in zip(c1s, c2s)]
    idxs, _ = _greedy_multi(ioxs, kth, limits3, 11, 0)

    for g in range(groups):
        v = jnp.bitwise_or(jnp.left_shift(hps[g] - 32768, 16), lps[g])
        sv = jnp.where(v >= 0, v,
                       jnp.bitwise_not(jnp.bitwise_xor(v, i32min)))
        val_ref[g * gr:(g + 1) * gr, :] = lax.bitcast_convert_type(
            sv, jnp.float32)
        idx_ref[g * gr:(g + 1) * gr, :] = idxs[g]


def _median_2d(x2, block_rows):
    m, n = x2.shape
    kth = (n - 1) // 2
    grid = (m // block_rows,)
    vals, idx = pl.pallas_call(
        functools.partial(_median_body, kth=kth, groups=2),
        grid=grid,
        in_specs=[pl.BlockSpec((block_rows, n), lambda j: (j, 0))],
        out_specs=[
            pl.BlockSpec((block_rows, 1), lambda j: (j, 0)),
            pl.BlockSpec((block_rows, 1), lambda j: (j, 0)),
        ],
        out_shape=[
            jax.ShapeDtypeStruct((m, 1), jnp.float32),
            jax.ShapeDtypeStruct((m, 1), jnp.int32),
        ],
    )(x2)
    return vals[:, 0], idx[:, 0]


_N = 2048
_KTH = (_N - 1) // 2
_NVEC = _N // 16
_CHUNK = 16


def _keys16(xv):
    i32min = jnp.int32(-(2 ** 31))
    s = plsc.bitcast(xv, jnp.int32)
    return jnp.where(s >= 0, s, jnp.bitwise_xor(jnp.bitwise_not(s), i32min))


def _make_sc_median(m_sc):
    info = plsc.get_sparse_core_info()
    nworkers = info.num_cores * info.num_subcores
    rpw = m_sc // nworkers
    assert m_sc % (nworkers * _CHUNK) == 0
    mesh = plsc.VectorSubcoreMesh(core_axis_name="c", subcore_axis_name="s")

    @functools.partial(
        pl.kernel,
        mesh=mesh,
        compiler_params=pltpu.CompilerParams(needs_layout_passes=False),
        out_type=[
            jax.ShapeDtypeStruct((m_sc,), jnp.float32),
            jax.ShapeDtypeStruct((m_sc,), jnp.int32),
        ],
        scratch_types=[
            pltpu.VMEM((_CHUNK * _N,), jnp.float32),
            pltpu.VMEM((512,), jnp.int32),
            pltpu.VMEM((_N + 16,), jnp.int32),
            pltpu.VMEM((_N + 16,), jnp.int32),
            pltpu.VMEM((rpw,), jnp.float32),
            pltpu.VMEM((rpw,), jnp.int32),
        ],
    )
    def sc_median(x_hbm, val_hbm, idx_hbm, xbuf, hist, ckey, cidx, oval, oidx):
        wid = lax.axis_index("s") * info.num_cores + lax.axis_index("c")
        base = wid * rpw
        zeros16 = jnp.zeros((16,), jnp.int32)
        ones16 = jnp.ones((16,), jnp.int32)
        lane = lax.iota(jnp.int32, 16)

        def zero_hist(j, _):
            hist[pl.ds(j * 16, 16)] = zeros16
            return 0

        lax.fori_loop(0, 512 // 16, zero_hist, 0)

        def do_chunk(ci, _):
            pltpu.sync_copy(
                x_hbm.at[pl.ds((base + ci * _CHUNK) * _N, _CHUNK * _N)],
                xbuf)

            def do_row(ri, acc):
                def pass_a(j4, _):
                    for u in range(4):
                        key = _keys16(
                            xbuf[pl.ds(ri * _N + (j4 * 4 + u) * 16, 16)])
                        b = jnp.right_shift(key, 23) + 256
                        plsc.addupdate_scatter(hist, [b], ones16)
                    return 0

                lax.fori_loop(0, _NVEC // 4, pass_a, 0)

                def locate(j, carry):
                    tot, bfound, cbelow = carry
                    hv = hist[pl.ds(j * 16, 16)]
                    hist[pl.ds(j * 16, 16)] = zeros16
                    cs = plsc.cumsum(hv)
                    cross = (tot + cs) > _KTH
                    has = jnp.any(cross)
                    first = jnp.max(plsc.all_reduce_ffs(cross))
                    excl = tot + cs - hv
                    cb = jnp.sum(jnp.where(lane == first, excl, 0))
                    newly = jnp.logical_and(has, bfound < 0)
                    bfound = jnp.where(newly, j * 16 + first, bfound)
                    cbelow = jnp.where(newly, cb, cbelow)
                    return tot + jnp.max(cs), bfound, cbelow

                _, bkt, cbelow = lax.fori_loop(
                    0, 512 // 16, locate,
                    (jnp.int32(0), jnp.int32(-1), jnp.int32(0)))
                b9 = bkt - 256

                def pass_b(j, cnt):
                    key = _keys16(xbuf[pl.ds(ri * _N + j * 16, 16)])
                    mask = jnp.right_shift(key, 23) == b9
                    plsc.store_compressed(ckey.at[pl.ds(cnt, 16)], key, mask=mask)
                    iv = j * 16 + lane
                    plsc.store_compressed(cidx.at[pl.ds(cnt, 16)], iv, mask=mask)
                    return cnt + jnp.max(
                        plsc.all_reduce_population_count(mask))

                cnt = lax.fori_loop(0, _NVEC, pass_b, jnp.int32(0))
                ckey[pl.ds(cnt, 16)] = jnp.full((16,), 2 ** 31 - 1, jnp.int32)
                nv = (cnt + 15) // 16

                bktbase = jnp.left_shift(b9, 23)

                def count_lt(trial):
                    def cbody(v, c):
                        kv = ckey[pl.ds(v * 16, 16)]
                        return c + jnp.sum((kv < trial).astype(jnp.int32))
                    return lax.fori_loop(0, nv, cbody, jnp.int32(0))

                p = jnp.int32(0)
                c_acc = jnp.int32(0)
                for bitpos in range(22, -1, -1):
                    cand = jnp.bitwise_or(p, jnp.int32(1 << bitpos))
                    c = count_lt(bktbase + cand)
                    ok = cbelow + c <= _KTH
                    p = jnp.where(ok, cand, p)
                    c_acc = jnp.where(ok, c, c_acc)
                vkey = bktbase + p
                r_total = _KTH - cbelow - c_acc

                def idx_scan(v, carry):
                    cum, fidx = carry
                    kv = ckey[pl.ds(v * 16, 16)]
                    iv = cidx[pl.ds(v * 16, 16)]
                    eq = kv == vkey
                    cs = plsc.cumsum(eq.astype(jnp.int32))
                    hitm = jnp.logical_and(eq, cs == r_total - cum + 1)
                    fidx = fidx + jnp.sum(jnp.where(hitm, iv, 0))
                    return cum + jnp.sum(eq.astype(jnp.int32)), fidx

                _, med_idx = lax.fori_loop(0, nv, idx_scan,
                                           (jnp.int32(0), jnp.int32(0)))
                mv16 = xbuf[pl.ds(ri * _N + med_idx - med_idx % 16, 16)]
                med_val = jnp.sum(jnp.where(lane == med_idx % 16, mv16, 0.0))
                vvec, ivec = acc
                sel = lane == ri
                vvec = jnp.where(sel, jnp.full((16,), med_val, jnp.float32),
                                 vvec)
                ivec = jnp.where(sel, jnp.full((16,), med_idx, jnp.int32),
                                 ivec)
                return vvec, ivec

            vvec, ivec = lax.fori_loop(
                0, _CHUNK, do_row,
                (jnp.zeros((16,), jnp.float32), jnp.zeros((16,), jnp.int32)))
            oval[pl.ds(ci * _CHUNK, 16)] = vvec
            oidx[pl.ds(ci * _CHUNK, 16)] = ivec
            return 0

        lax.fori_loop(0, rpw // _CHUNK, do_chunk, 0)
        pltpu.sync_copy(oval, val_hbm.at[pl.ds(base, rpw)])
        pltpu.sync_copy(oidx, idx_hbm.at[pl.ds(base, rpw)])

    return sc_median


_M_SC = 3072


def kernel(x):
    b, s, n = x.shape
    m = b * s
    x2 = x.reshape(m, n)
    if m > _M_SC and (m - _M_SC) % 256 == 0:
        m_sc = _M_SC
        val_sc, idx_sc = _make_sc_median(m_sc)(x2[:m_sc].reshape(-1))
        val_tc, idx_tc = _median_2d(x2[m_sc:], 256)
        vals = jnp.concatenate([val_sc, val_tc])
        idx = jnp.concatenate([idx_sc, idx_tc])
    else:
        block_rows = 256 if m % 256 == 0 else m
        vals, idx = _median_2d(x2, block_rows)
    return vals.reshape(b, s), idx.reshape(b, s).astype(jnp.int64)
